# baseline (speedup 1.0000x reference)
import functools

import jax
import jax.numpy as jnp
from jax import lax
from jax.experimental import pallas as pl
from jax.experimental.pallas import tpu as pltpu
from jax.experimental.pallas import tpu_sc as plsc

_info = plsc.get_sparse_core_info()
_NC, _NS = _info.num_cores, _info.num_subcores
_NW = _NC * _NS


@functools.lru_cache(maxsize=None)
def _make_gather(B, D):
    assert B % (8 * _NW) == 0 and D % _info.num_lanes == 0
    bpw = B // _NW
    CH = min(bpw, 128)
    NP = bpw // CH
    mesh = plsc.VectorSubcoreMesh(core_axis_name="c", subcore_axis_name="s")

    @functools.partial(
        pl.kernel,
        mesh=mesh,
        out_type=[
            jax.ShapeDtypeStruct((B, D), jnp.float32),
            jax.ShapeDtypeStruct((B, D), jnp.float32),
        ],
        scratch_types=[
            pltpu.VMEM((bpw,), jnp.int32),
            pltpu.VMEM((bpw,), jnp.int32),
            pltpu.VMEM((2, CH, D), jnp.float32),
            pltpu.VMEM((2, CH, D), jnp.float32),
            pltpu.SemaphoreType.DMA,
            pltpu.SemaphoreType.DMA,
        ],
    )
    def gather(uidx_hbm, midx_hbm, utab_hbm, mtab_hbm, uout_hbm, mout_hbm,
               uidx_v, midx_v, urows_v, mrows_v, sem_u, sem_m):
        wid = lax.axis_index("s") * _NC + lax.axis_index("c")
        base = wid * bpw
        pltpu.sync_copy(uidx_hbm.at[pl.ds(base, bpw)], uidx_v)
        pltpu.sync_copy(midx_hbm.at[pl.ds(base, bpw)], midx_v)
        cps = [None, None]
        for p in range(NP + 1):
            sl = p % 2
            if p < NP:
                cu = pltpu.async_copy(
                    utab_hbm.at[uidx_v.at[pl.ds(p * CH, CH)]],
                    urows_v.at[sl], sem_u)
                cm = pltpu.async_copy(
                    mtab_hbm.at[midx_v.at[pl.ds(p * CH, CH)]],
                    mrows_v.at[sl], sem_m)
                cps[sl] = (cu, cm)
            if p > 0:
                pr = (p - 1) % 2
                cu_p, cm_p = cps[pr]
                cu_p.wait()
                cm_p.wait()
                off = base + (p - 1) * CH
                pltpu.sync_copy(urows_v.at[pr], uout_hbm.at[pl.ds(off, CH)])
                pltpu.sync_copy(mrows_v.at[pr], mout_hbm.at[pl.ds(off, CH)])

    return gather


def _mlp_body(u2_ref, m2_ref, pu_ref, pm_ref, w1u_ref, w1m_ref, b1_ref,
              w2t_ref, b2_ref, w3r_ref, b3_ref, out_ref):
    D = w1u_ref.shape[0]
    u2 = u2_ref[...]
    m2 = m2_ref[...]
    u = jnp.where(pu_ref[...] > 0.5, u2[:, D:], u2[:, :D])
    m = jnp.where(pm_ref[...] > 0.5, m2[:, D:], m2[:, :D])
    x = jnp.dot(u, w1u_ref[...], preferred_element_type=jnp.float32)
    x = x + jnp.dot(m, w1m_ref[...], preferred_element_type=jnp.float32)
    x = jnp.maximum(x + b1_ref[...], 0.0)
    x = jnp.maximum(
        jnp.dot(x, w2t_ref[...], preferred_element_type=jnp.float32) + b2_ref[...],
        0.0)
    out_ref[...] = jnp.sum(x * w3r_ref[...], axis=1, keepdims=True) + b3_ref[...]


def kernel(user, movie, user_table, movie_table, W1, b1, W2, b2, W3, b3):
    B = user.shape[0]
    D = user_table.shape[1]
    H1 = W1.shape[0]
    H2 = W2.shape[0]

    user = user.astype(jnp.int32)
    movie = movie.astype(jnp.int32)
    ut2 = user_table.reshape(user_table.shape[0] // 2, 2 * D)
    mt2 = movie_table.reshape(movie_table.shape[0] // 2, 2 * D)

    u_rows, m_rows = _make_gather(B, 2 * D)(
        user >> 1, movie >> 1, ut2, mt2)

    pu = (user & 1).astype(jnp.float32).reshape(B, 1)
    pm = (movie & 1).astype(jnp.float32).reshape(B, 1)

    w1u = W1[:, :D].T
    w1m = W1[:, D:].T
    w2t = W2.T
    w3r = W3.reshape(1, H2)
    b1r = b1.reshape(1, H1)
    b2r = b2.reshape(1, H2)
    b3r = b3.reshape(1, 1)

    BLK = 2048
    out = pl.pallas_call(
        _mlp_body,
        grid=(B // BLK,),
        in_specs=[
            pl.BlockSpec((BLK, 2 * D), lambda i: (i, 0)),
            pl.BlockSpec((BLK, 2 * D), lambda i: (i, 0)),
            pl.BlockSpec((BLK, 1), lambda i: (i, 0)),
            pl.BlockSpec((BLK, 1), lambda i: (i, 0)),
            pl.BlockSpec((D, H1), lambda i: (0, 0)),
            pl.BlockSpec((D, H1), lambda i: (0, 0)),
            pl.BlockSpec((1, H1), lambda i: (0, 0)),
            pl.BlockSpec((H1, H2), lambda i: (0, 0)),
            pl.BlockSpec((1, H2), lambda i: (0, 0)),
            pl.BlockSpec((1, H2), lambda i: (0, 0)),
            pl.BlockSpec((1, 1), lambda i: (0, 0)),
        ],
        out_specs=pl.BlockSpec((BLK, 1), lambda i: (i, 0)),
        out_shape=jax.ShapeDtypeStruct((B, 1), jnp.float32),
    )(u_rows, m_rows, pu, pm, w1u, w1m, b1r, w2t, b2r, w3r, b3r)
    return out
